# baseline (speedup 1.0000x reference)
import jax
import jax.numpy as jnp
from jax import lax
from jax.experimental import pallas as pl
from jax.experimental.pallas import tpu as pltpu
from jax.experimental.pallas import tpu_sc as plsc

N_NODES = 10000
N_EDGES = 160000
D = 256
DH = 128
NS = 16
CH = 40
EPT = N_EDGES // NS
NCHUNK = EPT // CH
NR = 4
NI = 8
DLOOK = NR - 1
ILOOK = NI - 2
ROWS_PT = 624
TAIL_R0 = NS * ROWS_PT
TAIL_ROWS = N_NODES - TAIL_R0


def _gine_body(feat, src, dst, efeat, out,
               sidx, didx, grows, erows, acc, *sems):
    isem = sems[:NI]
    gsem = sems[NI:NI + NR]
    ssem = sems[NI + NR:NI + NR + NR]
    nsem = sems[NI + NR + NR]
    c = lax.axis_index("c")
    s = lax.axis_index("s")
    r0 = s * ROWS_PT
    pltpu.async_copy(feat.at[pl.ds(r0, ROWS_PT), pl.ds(c * DH, DH)],
                     acc.at[pl.ds(r0, ROWS_PT)], nsem)

    @pl.when(s == NS - 1)
    def _init_tail():
        pltpu.async_copy(feat.at[pl.ds(TAIL_R0, TAIL_ROWS),
                                 pl.ds(c * DH, DH)],
                         acc.at[pl.ds(TAIL_R0, TAIL_ROWS)], nsem)

    ebase = s * EPT

    def prep_idx(j, ri):
        pltpu.async_copy(src.at[pl.ds(ebase + j * CH, CH)],
                         sidx.at[ri], isem[ri])
        pltpu.async_copy(dst.at[pl.ds(ebase + j * CH, CH)],
                         didx.at[ri], isem[ri])

    def prep_data(j, r, ri, first):
        if not first:
            pltpu.make_async_copy(feat.at[pl.ds(0, CH), pl.ds(0, DH)],
                                  grows.at[r], ssem[r]).wait()
        pltpu.make_async_copy(src.at[pl.ds(0, CH)], sidx.at[ri],
                              isem[ri]).wait()
        pltpu.make_async_copy(src.at[pl.ds(0, CH)], didx.at[ri],
                              isem[ri]).wait()
        pltpu.async_copy(feat.at[sidx.at[ri], pl.ds(c * DH, DH)],
                         grows.at[r], gsem[r])
        pltpu.async_copy(efeat.at[pl.ds(ebase + j * CH, CH),
                                  pl.ds(c * DH, DH)],
                         erows.at[r], gsem[r])

    def proc(r, ri):
        pltpu.make_async_copy(feat.at[pl.ds(0, CH), pl.ds(0, DH)],
                              grows.at[r], gsem[r]).wait()
        pltpu.make_async_copy(feat.at[pl.ds(0, CH), pl.ds(0, DH)],
                              erows.at[r], gsem[r]).wait()

        def edge(e, c2):
            for f in range(DH // 16):
                sl = pl.ds(f * 16, 16)
                grows[r, e, sl] = jnp.maximum(grows[r, e, sl] + erows[r, e, sl],
                                              0.0)
            return c2

        lax.fori_loop(0, CH, edge, 0)
        pltpu.async_copy(grows.at[r], acc.at[didx.at[ri]], ssem[r], add=True)

    def substep(j, u, first_data, do_data, do_idx):
        if do_data:
            prep_data(j + DLOOK, (u + DLOOK) % NR, (u + DLOOK) % NI,
                      first_data)
        if do_idx:
            prep_idx(j + ILOOK, (u + ILOOK) % NI)
        proc(u % NR, u % NI)

    for j in range(ILOOK):
        prep_idx(j, j % NI)
    for j in range(DLOOK):
        prep_data(j, j % NR, j % NI, first=True)
    pltpu.make_async_copy(feat.at[pl.ds(r0, ROWS_PT), pl.ds(c * DH, DH)],
                          acc.at[pl.ds(r0, ROWS_PT)], nsem).wait()

    @pl.when(s == NS - 1)
    def _init_tail_wait():
        pltpu.make_async_copy(feat.at[pl.ds(0, TAIL_ROWS),
                                      pl.ds(c * DH, DH)],
                              acc.at[pl.ds(TAIL_R0, TAIL_ROWS)], nsem).wait()

    plsc.subcore_barrier()
    for j in range(NI):
        substep(j, j, first_data=(j + DLOOK < NR), do_data=True, do_idx=True)

    def steady(t, carry):
        j0 = t * NI
        for u in range(NI):
            substep(j0 + u, u, first_data=False, do_data=True, do_idx=True)
        return carry

    nsteady = (NCHUNK - ILOOK) // NI
    lax.fori_loop(1, nsteady, steady, 0)

    for j in range(nsteady * NI, NCHUNK):
        substep(j, j % NI, first_data=False,
                do_data=j + DLOOK < NCHUNK, do_idx=j + ILOOK < NCHUNK)
    for j in range(NCHUNK - NR, NCHUNK):
        pltpu.make_async_copy(feat.at[pl.ds(0, CH), pl.ds(0, DH)],
                              grows.at[j % NR], ssem[j % NR]).wait()

    plsc.subcore_barrier()
    pltpu.sync_copy(acc.at[pl.ds(r0, ROWS_PT)],
                    out.at[pl.ds(r0, ROWS_PT), pl.ds(c * DH, DH)])

    @pl.when(s == NS - 1)
    def _out_tail():
        pltpu.sync_copy(acc.at[pl.ds(TAIL_R0, TAIL_ROWS)],
                        out.at[pl.ds(TAIL_R0, TAIL_ROWS), pl.ds(c * DH, DH)])


def kernel(feat, edge_index, efeat):
    src = edge_index[0].astype(jnp.int32)
    dst = edge_index[1].astype(jnp.int32)
    mesh = plsc.VectorSubcoreMesh(core_axis_name="c", subcore_axis_name="s")
    k = pl.kernel(
        _gine_body,
        mesh=mesh,
        out_type=jax.ShapeDtypeStruct((N_NODES, D), jnp.float32),
        scratch_types=[
            pltpu.VMEM((NI, CH), jnp.int32),
            pltpu.VMEM((NI, CH), jnp.int32),
            pltpu.VMEM((NR, CH, DH), jnp.float32),
            pltpu.VMEM((NR, CH, DH), jnp.float32),
            pltpu.VMEM_SHARED((N_NODES, DH), jnp.float32),
        ] + [pltpu.SemaphoreType.DMA] * (NI + NR + NR + 1),
    )
    return k(feat, src, dst, efeat)


# speedup vs baseline: 1.0243x; 1.0243x over previous
import jax
import jax.numpy as jnp
from jax import lax
from jax.experimental import pallas as pl
from jax.experimental.pallas import tpu as pltpu
from jax.experimental.pallas import tpu_sc as plsc

N_NODES = 10000
N_EDGES = 160000
D = 256
DH = 128
NS = 16
CH = 40
EPT = N_EDGES // NS
NCHUNK = EPT // CH
NR = 4
NI = 8
DLOOK = NR - 1
ILOOK = NI - 2
ROWS_PT = 624
TAIL_R0 = NS * ROWS_PT
TAIL_ROWS = N_NODES - TAIL_R0


def _gine_body(fh, gidx, dst, efeat, out,
               sidx, didx, grows, erows, acc, *sems):
    isem = sems[:NI]
    gsem = sems[NI:NI + NR]
    ssem = sems[NI + NR:NI + NR + NR]
    nsem = sems[NI + NR + NR]
    c = lax.axis_index("c")
    s = lax.axis_index("s")
    r0 = s * ROWS_PT
    pltpu.async_copy(fh.at[pl.ds(c * N_NODES + r0, ROWS_PT)],
                     acc.at[pl.ds(r0, ROWS_PT)], nsem)

    @pl.when(s == NS - 1)
    def _init_tail():
        pltpu.async_copy(fh.at[pl.ds(c * N_NODES + TAIL_R0, TAIL_ROWS)],
                         acc.at[pl.ds(TAIL_R0, TAIL_ROWS)], nsem)

    ebase = s * EPT

    def prep_idx(j, ri):
        pltpu.async_copy(gidx.at[pl.ds(c * N_EDGES + ebase + j * CH, CH)],
                         sidx.at[ri], isem[ri])
        pltpu.async_copy(dst.at[pl.ds(ebase + j * CH, CH)],
                         didx.at[ri], isem[ri])

    def prep_data(j, r, ri, first):
        if not first:
            pltpu.make_async_copy(fh.at[pl.ds(0, CH)], grows.at[r],
                                  ssem[r]).wait()
        pltpu.make_async_copy(gidx.at[pl.ds(0, CH)], sidx.at[ri],
                              isem[ri]).wait()
        pltpu.make_async_copy(gidx.at[pl.ds(0, CH)], didx.at[ri],
                              isem[ri]).wait()
        pltpu.async_copy(fh.at[sidx.at[ri]], grows.at[r], gsem[r])
        pltpu.async_copy(efeat.at[pl.ds(ebase + j * CH, CH),
                                  pl.ds(c * DH, DH)],
                         erows.at[r], gsem[r])

    def proc(r, ri):
        pltpu.make_async_copy(fh.at[pl.ds(0, CH)], grows.at[r],
                              gsem[r]).wait()
        pltpu.make_async_copy(fh.at[pl.ds(0, CH)], erows.at[r],
                              gsem[r]).wait()

        def edge(e, c2):
            for f in range(DH // 16):
                sl = pl.ds(f * 16, 16)
                grows[r, e, sl] = jnp.maximum(grows[r, e, sl] + erows[r, e, sl],
                                              0.0)
            return c2

        lax.fori_loop(0, CH, edge, 0)
        pltpu.async_copy(grows.at[r], acc.at[didx.at[ri]], ssem[r], add=True)

    def substep(j, u, first_data, do_data, do_idx):
        if do_data:
            prep_data(j + DLOOK, (u + DLOOK) % NR, (u + DLOOK) % NI,
                      first_data)
        if do_idx:
            prep_idx(j + ILOOK, (u + ILOOK) % NI)
        proc(u % NR, u % NI)

    for j in range(ILOOK):
        prep_idx(j, j % NI)
    for j in range(DLOOK):
        prep_data(j, j % NR, j % NI, first=True)
    pltpu.make_async_copy(fh.at[pl.ds(c * N_NODES + r0, ROWS_PT)],
                          acc.at[pl.ds(r0, ROWS_PT)], nsem).wait()

    @pl.when(s == NS - 1)
    def _init_tail_wait():
        pltpu.make_async_copy(fh.at[pl.ds(0, TAIL_ROWS)],
                              acc.at[pl.ds(TAIL_R0, TAIL_ROWS)], nsem).wait()

    plsc.subcore_barrier()
    for j in range(NI):
        substep(j, j, first_data=(j + DLOOK < NR), do_data=True, do_idx=True)

    def steady(t, carry):
        j0 = t * NI
        for u in range(NI):
            substep(j0 + u, u, first_data=False, do_data=True, do_idx=True)
        return carry

    nsteady = (NCHUNK - ILOOK) // NI
    lax.fori_loop(1, nsteady, steady, 0)

    for j in range(nsteady * NI, NCHUNK):
        substep(j, j % NI, first_data=False,
                do_data=j + DLOOK < NCHUNK, do_idx=j + ILOOK < NCHUNK)
    for j in range(NCHUNK - NR, NCHUNK):
        pltpu.make_async_copy(fh.at[pl.ds(0, CH)], grows.at[j % NR],
                              ssem[j % NR]).wait()

    plsc.subcore_barrier()
    pltpu.sync_copy(acc.at[pl.ds(r0, ROWS_PT)],
                    out.at[pl.ds(r0, ROWS_PT), pl.ds(c * DH, DH)])

    @pl.when(s == NS - 1)
    def _out_tail():
        pltpu.sync_copy(acc.at[pl.ds(TAIL_R0, TAIL_ROWS)],
                        out.at[pl.ds(TAIL_R0, TAIL_ROWS), pl.ds(c * DH, DH)])


def kernel(feat, edge_index, efeat):
    src = edge_index[0].astype(jnp.int32)
    dst = edge_index[1].astype(jnp.int32)
    gidx = jnp.concatenate([src, src + N_NODES])
    fh = jnp.concatenate([feat[:, :DH], feat[:, DH:]], axis=0)
    mesh = plsc.VectorSubcoreMesh(core_axis_name="c", subcore_axis_name="s")
    k = pl.kernel(
        _gine_body,
        mesh=mesh,
        out_type=jax.ShapeDtypeStruct((N_NODES, D), jnp.float32),
        scratch_types=[
            pltpu.VMEM((NI, CH), jnp.int32),
            pltpu.VMEM((NI, CH), jnp.int32),
            pltpu.VMEM((NR, CH, DH), jnp.float32),
            pltpu.VMEM((NR, CH, DH), jnp.float32),
            pltpu.VMEM_SHARED((N_NODES, DH), jnp.float32),
        ] + [pltpu.SemaphoreType.DMA] * (NI + NR + NR + 1),
    )
    return k(fh, gidx, dst, efeat)
